# Initial kernel scaffold; baseline (speedup 1.0000x reference)
import jax
import jax.numpy as jnp
from jax.experimental import pallas as pl


def kernel(tokens, q_k):
    raise NotImplementedError("write your pallas kernel here")


# speedup vs baseline: 1.1490x; 1.1490x over previous
import math

import jax
import jax.numpy as jnp
from jax.experimental import pallas as pl

_B, _N, _C, _S = 16, 4096, 128, 32
_EPS = 1e-12


def _vlaq_kernel(tokens_ref, qk_ref, out_ref):
    z = tokens_ref[0]
    qk = qk_ref[...]
    scale = 1.0 / math.sqrt(_C)
    scores = jax.lax.dot_general(
        z, qk, (((1,), (1,)), ((), ())),
        preferred_element_type=jnp.float32) * scale
    m = jnp.max(scores, axis=0, keepdims=True)
    e = jnp.exp(scores - m)
    denom = jnp.sum(e, axis=0)
    agg = jax.lax.dot_general(
        e, z, (((0,), (0,)), ((), ())),
        preferred_element_type=jnp.float32)
    resid = agg / denom[:, None] - qk
    rown = jnp.sqrt(jnp.sum(resid * resid, axis=1, keepdims=True))
    r1 = resid / jnp.maximum(rown, _EPS)
    tot = jnp.sqrt(jnp.sum(r1 * r1))
    out_ref[0] = r1 / jnp.maximum(tot, _EPS)


def kernel(tokens, q_k):
    b, n, c = tokens.shape
    s = q_k.shape[0]
    out = pl.pallas_call(
        _vlaq_kernel,
        grid=(b,),
        in_specs=[
            pl.BlockSpec((1, n, c), lambda i: (i, 0, 0)),
            pl.BlockSpec((s, c), lambda i: (0, 0)),
        ],
        out_specs=pl.BlockSpec((1, s, c), lambda i: (i, 0, 0)),
        out_shape=jax.ShapeDtypeStruct((b, s, c), jnp.float32),
    )(tokens, q_k)
    return out.reshape(b, s * c)
